# baseline (speedup 1.0000x reference)
import functools

import jax
import jax.numpy as jnp
from jax import lax
from jax.experimental import pallas as pl
from jax.experimental.pallas import tpu as pltpu
from jax.experimental.pallas import tpu_sc as plsc

C = 21
N = 8
H = 512
W = 512
NC = 2
NS = 16
NW = NC * NS
ROWS_PER_TILE = (N * H) // NW
K = 4
SLABS = ROWS_PER_TILE // K
PIX = K * W
VECS = PIX // 16

_LN2 = 0.6931471805599453
_SQRT2 = 1.4142135623730951


def _log_f32(p):
    bits = plsc.bitcast(p, jnp.int32)
    e = (bits >> 23) - 127
    m = plsc.bitcast((bits & 0x007FFFFF) | 0x3F800000, jnp.float32)
    big = m > _SQRT2
    m = jnp.where(big, m * 0.5, m)
    ef = jnp.where(big, e + 1, e).astype(jnp.float32)
    r = (m - 1.0) / (m + 1.0)
    r2 = r * r
    poly = r * (2.0 + r2 * (0.6666666666666666 + r2 * (0.4 + r2 * (2.0 / 7.0))))
    return ef * _LN2 + poly


def _body(inp, tgt, alf, out, slab_v, tgt_v, alf_v, acc_v, slab_sem, tgt_sem):
    c = lax.axis_index("c")
    s = lax.axis_index("s")
    wid = s * NC + c
    n = wid // 4
    h_base = (wid % 4) * ROWS_PER_TILE

    pltpu.sync_copy(alf, alf_v)
    lane = lax.iota(jnp.int32, 16)

    def start(si, b):
        h0 = h_base + si * K
        pltpu.async_copy(
            inp.at[n, :, pl.ds(h0 * W, PIX)], slab_v.at[b], slab_sem.at[b]
        )
        pltpu.async_copy(
            tgt.at[pl.ds((n * H + h0) * W, PIX)], tgt_v.at[b], tgt_sem.at[b]
        )

    start(0, 0)
    start(1, 1)

    def pair_loop(g, acc):
        for b in range(2):
            si = g * 2 + b
            pltpu.make_async_copy(
                inp.at[n, :, pl.ds(0, PIX)], slab_v.at[b], slab_sem.at[b]
            ).wait()
            pltpu.make_async_copy(
                tgt.at[pl.ds(0, PIX)], tgt_v.at[b], tgt_sem.at[b]
            ).wait()

            def vec_loop(j, a_in, b=b):
                base = j * 16
                t = tgt_v[b, pl.ds(base, 16)]
                p = plsc.load_gather(slab_v.at[b], [t, base + lane]) + 1e-10
                a = plsc.load_gather(alf_v, [t])
                omp = 1.0 - p
                return a_in - a * omp * omp * _log_f32(p)

            acc = lax.fori_loop(0, VECS, vec_loop, acc, unroll=8)

            @pl.when(si + 2 < SLABS)
            def _():
                start(si + 2, b)

        return acc

    acc = lax.fori_loop(0, SLABS // 2, pair_loop, jnp.zeros((16,), jnp.float32))
    acc_v[...] = acc
    pltpu.sync_copy(acc_v, out.at[wid])


@jax.jit
def _focal_partials(inp3, tgt1, alf1):
    mesh = plsc.VectorSubcoreMesh(core_axis_name="c", subcore_axis_name="s")
    return pl.kernel(
        _body,
        out_type=jax.ShapeDtypeStruct((NW, 16), jnp.float32),
        mesh=mesh,
        compiler_params=pltpu.CompilerParams(
            use_tc_tiling_on_sc=False, needs_layout_passes=False
        ),
        scratch_types=[
            pltpu.VMEM((2, C, PIX), jnp.float32),
            pltpu.VMEM((2, PIX), jnp.int32),
            pltpu.VMEM((C,), jnp.float32),
            pltpu.VMEM((16,), jnp.float32),
            pltpu.SemaphoreType.DMA((2,)),
            pltpu.SemaphoreType.DMA((2,)),
        ],
    )(inp3, tgt1, alf1)


def kernel(input, target, alpha, one_hot_codes):
    inp3 = input.reshape(N, C, H * W)
    tgt1 = target.reshape(-1).astype(jnp.int32)
    alf1 = alpha.reshape(-1)
    partials = _focal_partials(inp3, tgt1, alf1)
    return jnp.sum(partials) / (N * H * W)


# speedup vs baseline: 1.0036x; 1.0036x over previous
import functools

import jax
import jax.numpy as jnp
from jax import lax
from jax.experimental import pallas as pl
from jax.experimental.pallas import tpu as pltpu
from jax.experimental.pallas import tpu_sc as plsc

C = 21
N = 8
H = 512
W = 512
NC = 2
NS = 16
NW = NC * NS
ROWS_PER_TILE = (N * H) // NW
K = 4
SLABS = ROWS_PER_TILE // K
PIX = K * W
VECS = PIX // 16

_LN2 = 0.6931471805599453
_SQRT2 = 1.4142135623730951


def _log_f32(p):
    bits = plsc.bitcast(p, jnp.int32)
    e = (bits >> 23) - 127
    m = plsc.bitcast((bits & 0x007FFFFF) | 0x3F800000, jnp.float32)
    big = m > _SQRT2
    m = jnp.where(big, m * 0.5, m)
    ef = jnp.where(big, e + 1, e).astype(jnp.float32)
    r = (m - 1.0) / (m + 1.0)
    r2 = r * r
    poly = r * (2.0 + r2 * (0.6666666666666666 + r2 * (0.4 + r2 * (2.0 / 7.0))))
    return ef * _LN2 + poly


def _body(inp, tgt, alf, out, slab_v, tgt_v, alf_v, acc_v, slab_sem, tgt_sem):
    c = lax.axis_index("c")
    s = lax.axis_index("s")
    wid = s * NC + c
    n = wid // 4
    h_base = (wid % 4) * ROWS_PER_TILE

    pltpu.sync_copy(alf, alf_v)
    lane = lax.iota(jnp.int32, 16)

    def start(si, b):
        h0 = h_base + si * K
        for q in range(3):
            pltpu.async_copy(
                inp.at[n, pl.ds(7 * q, 7), pl.ds(h0 * W, PIX)],
                slab_v.at[b, pl.ds(7 * q, 7)],
                slab_sem.at[b],
            )
        pltpu.async_copy(
            tgt.at[pl.ds((n * H + h0) * W, PIX)], tgt_v.at[b], tgt_sem.at[b]
        )

    start(0, 0)
    start(1, 1)

    def pair_loop(g, acc):
        for b in range(2):
            si = g * 2 + b
            for q in range(3):
                pltpu.make_async_copy(
                    inp.at[n, pl.ds(7 * q, 7), pl.ds(0, PIX)],
                    slab_v.at[b, pl.ds(7 * q, 7)],
                    slab_sem.at[b],
                ).wait()
            pltpu.make_async_copy(
                tgt.at[pl.ds(0, PIX)], tgt_v.at[b], tgt_sem.at[b]
            ).wait()

            def vec_loop(j, a_in, b=b):
                base = j * 16
                t = tgt_v[b, pl.ds(base, 16)]
                p = plsc.load_gather(slab_v.at[b], [t, base + lane]) + 1e-10
                a = plsc.load_gather(alf_v, [t])
                omp = 1.0 - p
                return a_in - a * omp * omp * _log_f32(p)

            acc = lax.fori_loop(0, VECS, vec_loop, acc, unroll=8)

            @pl.when(si + 2 < SLABS)
            def _():
                start(si + 2, b)

        return acc

    acc = lax.fori_loop(0, SLABS // 2, pair_loop, jnp.zeros((16,), jnp.float32))
    acc_v[...] = acc
    pltpu.sync_copy(acc_v, out.at[wid])


@jax.jit
def _focal_partials(inp3, tgt1, alf1):
    mesh = plsc.VectorSubcoreMesh(core_axis_name="c", subcore_axis_name="s")
    return pl.kernel(
        _body,
        out_type=jax.ShapeDtypeStruct((NW, 16), jnp.float32),
        mesh=mesh,
        compiler_params=pltpu.CompilerParams(
            use_tc_tiling_on_sc=False, needs_layout_passes=False
        ),
        scratch_types=[
            pltpu.VMEM((2, C, PIX), jnp.float32),
            pltpu.VMEM((2, PIX), jnp.int32),
            pltpu.VMEM((C,), jnp.float32),
            pltpu.VMEM((16,), jnp.float32),
            pltpu.SemaphoreType.DMA((2,)),
            pltpu.SemaphoreType.DMA((2,)),
        ],
    )(inp3, tgt1, alf1)


def kernel(input, target, alpha, one_hot_codes):
    inp3 = input.reshape(N, C, H * W)
    tgt1 = target.reshape(-1).astype(jnp.int32)
    alf1 = alpha.reshape(-1)
    partials = _focal_partials(inp3, tgt1, alf1)
    return jnp.sum(partials) / (N * H * W)


# speedup vs baseline: 2.1853x; 2.1775x over previous
import functools

import jax
import jax.numpy as jnp
from jax import lax
from jax.experimental import pallas as pl
from jax.experimental.pallas import tpu as pltpu
from jax.experimental.pallas import tpu_sc as plsc

C = 21
N = 8
H = 512
W = 512
NC = 2
NS = 16
NW = NC * NS
TH = 8
TW = 128
HT = H // TH
WT = W // TW
SLABS_TOTAL = N * HT * WT
SLABS = SLABS_TOTAL // NW
PIX = TH * TW
VECS = PIX // 16

_LN2 = 0.6931471805599453
_SQRT2 = 1.4142135623730951


def _log_f32(p):
    bits = plsc.bitcast(p, jnp.int32)
    e = (bits >> 23) - 127
    m = plsc.bitcast((bits & 0x007FFFFF) | 0x3F800000, jnp.float32)
    big = m > _SQRT2
    m = jnp.where(big, m * 0.5, m)
    ef = jnp.where(big, e + 1, e).astype(jnp.float32)
    r = (m - 1.0) / (m + 1.0)
    r2 = r * r
    poly = r * (2.0 + r2 * (0.6666666666666666 + r2 * (0.4 + r2 * (2.0 / 7.0))))
    return ef * _LN2 + poly


def _body(inp, tgt, alf, out, slab_v, tgt_v, alf_v, acc_v, slab_sem, tgt_sem):
    c = lax.axis_index("c")
    s = lax.axis_index("s")
    wid = s * NC + c
    f_base = wid * SLABS

    pltpu.sync_copy(alf, alf_v)
    lane = lax.iota(jnp.int32, 16)

    def start(si, b):
        f = f_base + si
        n = f // (HT * WT)
        rem = f % (HT * WT)
        h0 = (rem // WT) * TH
        w0 = (rem % WT) * TW
        pltpu.async_copy(
            inp.at[n, :, pl.ds(h0, TH), pl.ds(w0, TW)],
            slab_v.at[b],
            slab_sem.at[b],
        )
        pltpu.async_copy(
            tgt.at[n, pl.ds(h0, TH), pl.ds(w0, TW)], tgt_v.at[b], tgt_sem.at[b]
        )

    start(0, 0)
    start(1, 1)

    def pair_loop(g, acc):
        for b in range(2):
            si = g * 2 + b
            pltpu.make_async_copy(
                inp.at[0, :, pl.ds(0, TH), pl.ds(0, TW)],
                slab_v.at[b],
                slab_sem.at[b],
            ).wait()
            pltpu.make_async_copy(
                tgt.at[0, pl.ds(0, TH), pl.ds(0, TW)], tgt_v.at[b], tgt_sem.at[b]
            ).wait()

            def vec_loop(j, a_in, b=b):
                hi = j >> 3
                wv = (j & 7) * 16
                t = tgt_v[b, hi, pl.ds(wv, 16)]
                h_vec = jnp.full((16,), hi, jnp.int32)
                p = plsc.load_gather(slab_v.at[b], [t, h_vec, wv + lane]) + 1e-10
                a = plsc.load_gather(alf_v, [t])
                omp = 1.0 - p
                return a_in - a * omp * omp * _log_f32(p)

            acc = lax.fori_loop(0, VECS, vec_loop, acc, unroll=4)

            @pl.when(si + 2 < SLABS)
            def _():
                start(si + 2, b)

        return acc

    acc = lax.fori_loop(0, SLABS // 2, pair_loop, jnp.zeros((16,), jnp.float32))
    acc_v[...] = acc
    pltpu.sync_copy(acc_v, out.at[pl.ds(wid * 16, 16)])


@jax.jit
def _focal_partials(inp, tgt, alf1):
    mesh = plsc.VectorSubcoreMesh(core_axis_name="c", subcore_axis_name="s")
    return pl.kernel(
        _body,
        out_type=jax.ShapeDtypeStruct((NW * 16,), jnp.float32),
        mesh=mesh,
        compiler_params=pltpu.CompilerParams(needs_layout_passes=False),
        scratch_types=[
            pltpu.VMEM((2, C, TH, TW), jnp.float32),
            pltpu.VMEM((2, TH, TW), jnp.int32),
            pltpu.VMEM((C,), jnp.float32),
            pltpu.VMEM((16,), jnp.float32),
            pltpu.SemaphoreType.DMA((2,)),
            pltpu.SemaphoreType.DMA((2,)),
        ],
    )(inp, tgt, alf1)


def kernel(input, target, alpha, one_hot_codes):
    partials = _focal_partials(input, target.astype(jnp.int32), alpha.reshape(-1))
    return jnp.sum(partials) / (N * H * W)


# speedup vs baseline: 2.5173x; 1.1519x over previous
import functools

import jax
import jax.numpy as jnp
from jax import lax
from jax.experimental import pallas as pl
from jax.experimental.pallas import tpu as pltpu
from jax.experimental.pallas import tpu_sc as plsc

C = 21
N = 8
H = 512
W = 512
NC = 2
NS = 16
NW = NC * NS
TH = 8
TW = 128
HT = H // TH
WT = W // TW
SLABS_TOTAL = N * HT * WT
SLABS = SLABS_TOTAL // NW
PIX = TH * TW
VECS = PIX // 16
NBUF = 4

_LN2 = 0.6931471805599453
_SQRT2 = 1.4142135623730951


def _log_f32(p):
    bits = plsc.bitcast(p, jnp.int32)
    e = (bits >> 23) - 127
    m = plsc.bitcast((bits & 0x007FFFFF) | 0x3F800000, jnp.float32)
    big = m > _SQRT2
    m = jnp.where(big, m * 0.5, m)
    ef = jnp.where(big, e + 1, e).astype(jnp.float32)
    r = (m - 1.0) / (m + 1.0)
    r2 = r * r
    poly = r * (2.0 + r2 * (0.6666666666666666 + r2 * (0.4 + r2 * (2.0 / 7.0))))
    return ef * _LN2 + poly


def _body(inp, tgt, alf, out, slab_v, tgt_v, alf_v, acc_v, slab_sem, tgt_sem):
    c = lax.axis_index("c")
    s = lax.axis_index("s")
    wid = s * NC + c
    f_base = wid * SLABS

    pltpu.sync_copy(alf, alf_v)
    lane = lax.iota(jnp.int32, 16)

    def start(si, b):
        f = f_base + si
        n = f // (HT * WT)
        rem = f % (HT * WT)
        h0 = (rem // WT) * TH
        w0 = (rem % WT) * TW
        pltpu.async_copy(
            inp.at[n, :, pl.ds(h0, TH), pl.ds(w0, TW)],
            slab_v.at[b],
            slab_sem.at[b],
        )
        pltpu.async_copy(
            tgt.at[n, pl.ds(h0, TH), pl.ds(w0, TW)], tgt_v.at[b], tgt_sem.at[b]
        )

    for b0 in range(NBUF):
        start(b0, b0)

    def pair_loop(g, acc):
        for b in range(NBUF):
            si = g * NBUF + b
            pltpu.make_async_copy(
                inp.at[0, :, pl.ds(0, TH), pl.ds(0, TW)],
                slab_v.at[b],
                slab_sem.at[b],
            ).wait()
            pltpu.make_async_copy(
                tgt.at[0, pl.ds(0, TH), pl.ds(0, TW)], tgt_v.at[b], tgt_sem.at[b]
            ).wait()

            def vec_loop(j, a_in, b=b):
                hi = j >> 3
                wv = (j & 7) * 16
                t = tgt_v[b, hi, pl.ds(wv, 16)]
                h_vec = jnp.full((16,), hi, jnp.int32)
                p = plsc.load_gather(slab_v.at[b], [t, h_vec, wv + lane]) + 1e-10
                a = plsc.load_gather(alf_v, [t])
                omp = 1.0 - p
                return a_in - a * omp * omp * _log_f32(p)

            acc = lax.fori_loop(0, VECS, vec_loop, acc, unroll=4)

            @pl.when(si + NBUF < SLABS)
            def _():
                start(si + NBUF, b)

        return acc

    acc = lax.fori_loop(0, SLABS // NBUF, pair_loop, jnp.zeros((16,), jnp.float32))
    acc_v[...] = acc
    pltpu.sync_copy(acc_v, out.at[pl.ds(wid * 16, 16)])


@jax.jit
def _focal_partials(inp, tgt, alf1):
    mesh = plsc.VectorSubcoreMesh(core_axis_name="c", subcore_axis_name="s")
    return pl.kernel(
        _body,
        out_type=jax.ShapeDtypeStruct((NW * 16,), jnp.float32),
        mesh=mesh,
        compiler_params=pltpu.CompilerParams(needs_layout_passes=False),
        scratch_types=[
            pltpu.VMEM((NBUF, C, TH, TW), jnp.float32),
            pltpu.VMEM((NBUF, TH, TW), jnp.int32),
            pltpu.VMEM((C,), jnp.float32),
            pltpu.VMEM((16,), jnp.float32),
            pltpu.SemaphoreType.DMA((NBUF,)),
            pltpu.SemaphoreType.DMA((NBUF,)),
        ],
    )(inp, tgt, alf1)


def kernel(input, target, alpha, one_hot_codes):
    partials = _focal_partials(input, target.astype(jnp.int32), alpha.reshape(-1))
    return jnp.sum(partials) / (N * H * W)
